# baseline (speedup 1.0000x reference)
import functools

import jax
import jax.numpy as jnp
from jax import lax
from jax.experimental import pallas as pl
from jax.experimental.pallas import tpu as pltpu
from jax.experimental.pallas import tpu_sc as plsc

EMB = 64
CHUNK = 128


@functools.lru_cache(maxsize=None)
def _make_kernel(B, nc, ns):
    nw = nc * ns
    b_per_w = B // nw
    n_chunks = b_per_w // CHUNK
    mesh = plsc.VectorSubcoreMesh(core_axis_name="c", subcore_axis_name="s")

    @functools.partial(
        pl.kernel,
        mesh=mesh,
        compiler_params=pltpu.CompilerParams(use_tc_tiling_on_sc=False),
        out_type=jax.ShapeDtypeStruct((B, EMB), jnp.float32),
        scratch_types=[
            pltpu.VMEM((n_chunks, CHUNK), jnp.int32),
            pltpu.VMEM((CHUNK, EMB), jnp.float32),
            pltpu.SemaphoreType.DMA,
        ],
    )
    def emb_kernel(table_hbm, idx_hbm, out_hbm, idx_v, rows_v, sem):
        wid = lax.axis_index("s") * nc + lax.axis_index("c")
        base = wid * b_per_w
        pltpu.sync_copy(idx_hbm.at[wid], idx_v)

        def body(j, carry):
            pltpu.async_copy(table_hbm.at[idx_v.at[j]], rows_v, sem).wait()
            pltpu.sync_copy(rows_v, out_hbm.at[pl.ds(base + j * CHUNK, CHUNK)])
            return carry

        lax.fori_loop(0, n_chunks, body, 0)

    return emb_kernel


def kernel(glove_weight, word_indices):
    batch, seq = word_indices.shape
    B = batch * seq
    info = plsc.get_sparse_core_info()
    nc, ns = info.num_cores, info.num_subcores
    nw = nc * ns
    idx3 = word_indices.reshape(nw, B // (nw * CHUNK), CHUNK)
    out = _make_kernel(B, nc, ns)(glove_weight, idx3)
    return out.reshape(batch, seq, EMB)


# speedup vs baseline: 1.1177x; 1.1177x over previous
import functools

import jax
import jax.numpy as jnp
from jax import lax
from jax.experimental import pallas as pl
from jax.experimental.pallas import tpu as pltpu
from jax.experimental.pallas import tpu_sc as plsc

EMB = 64
CHUNK = 128
K = 4


@functools.lru_cache(maxsize=None)
def _make_kernel(B, nc, ns):
    nw = nc * ns
    b_per_w = B // nw
    n_chunks = b_per_w // CHUNK
    group = CHUNK * K
    ng = b_per_w // group
    assert b_per_w % group == 0 and ng % 2 == 0
    mesh = plsc.VectorSubcoreMesh(core_axis_name="c", subcore_axis_name="s")

    @functools.partial(
        pl.kernel,
        mesh=mesh,
        compiler_params=pltpu.CompilerParams(use_tc_tiling_on_sc=False),
        out_type=jax.ShapeDtypeStruct((B, EMB), jnp.float32),
        scratch_types=[
            pltpu.VMEM((n_chunks, CHUNK), jnp.int32),
            pltpu.VMEM((group, EMB), jnp.float32),
            pltpu.VMEM((group, EMB), jnp.float32),
            pltpu.SemaphoreType.DMA,
            pltpu.SemaphoreType.DMA,
        ],
    )
    def emb_kernel(table_hbm, idx_hbm, out_hbm, idx_v, buf_a, buf_b, sem_a, sem_b):
        wid = lax.axis_index("s") * nc + lax.axis_index("c")
        base = wid * b_per_w
        pltpu.sync_copy(idx_hbm.at[wid], idx_v)

        def fire(g, buf, sem):
            for b in range(K):
                pltpu.async_copy(
                    table_hbm.at[idx_v.at[g * K + b]],
                    buf.at[pl.ds(b * CHUNK, CHUNK)],
                    sem,
                )

        def drain(buf, sem):
            pltpu.make_async_copy(out_hbm.at[pl.ds(0, group)], buf, sem).wait()

        def store(g, buf):
            pltpu.sync_copy(buf, out_hbm.at[pl.ds(base + g * group, group)])

        fire(0, buf_a, sem_a)

        def body(i, carry):
            g0 = 2 * i
            g1 = g0 + 1
            fire(g1, buf_b, sem_b)
            drain(buf_a, sem_a)
            store(g0, buf_a)
            fire(jnp.minimum(g0 + 2, ng - 1), buf_a, sem_a)
            drain(buf_b, sem_b)
            store(g1, buf_b)
            return carry

        lax.fori_loop(0, ng // 2, body, 0)
        drain(buf_a, sem_a)

    return emb_kernel


def kernel(glove_weight, word_indices):
    batch, seq = word_indices.shape
    B = batch * seq
    info = plsc.get_sparse_core_info()
    nc, ns = info.num_cores, info.num_subcores
    nw = nc * ns
    idx3 = word_indices.reshape(nw, B // (nw * CHUNK), CHUNK)
    out = _make_kernel(B, nc, ns)(glove_weight, idx3)
    return out.reshape(batch, seq, EMB)


# speedup vs baseline: 1.1191x; 1.0012x over previous
import functools

import jax
import jax.numpy as jnp
from jax import lax
from jax.experimental import pallas as pl
from jax.experimental.pallas import tpu as pltpu
from jax.experimental.pallas import tpu_sc as plsc

EMB = 64
GB = 2


@functools.lru_cache(maxsize=None)
def _make_kernel(batch, seq, nc, ns):
    nw = nc * ns
    assert batch % nw == 0 and seq == 200
    bw = batch // nw
    b_per_w = bw * seq
    ng = bw // GB
    assert bw % GB == 0 and ng % 2 == 0
    row_chunks = [(0, 128), (128, 72)]
    mesh = plsc.VectorSubcoreMesh(core_axis_name="c", subcore_axis_name="s")

    @functools.partial(
        pl.kernel,
        mesh=mesh,
        compiler_params=pltpu.CompilerParams(use_tc_tiling_on_sc=False),
        out_type=jax.ShapeDtypeStruct((batch, seq, EMB), jnp.float32),
        scratch_types=[
            pltpu.VMEM((b_per_w,), jnp.int32),
            pltpu.VMEM((GB, seq, EMB), jnp.float32),
            pltpu.VMEM((GB, seq, EMB), jnp.float32),
            pltpu.SemaphoreType.DMA,
            pltpu.SemaphoreType.DMA,
        ],
    )
    def emb_kernel(table_hbm, idx_hbm, out_hbm, idx_v, buf_a, buf_b, sem_a, sem_b):
        wid = lax.axis_index("s") * nc + lax.axis_index("c")
        pltpu.sync_copy(idx_hbm.at[pl.ds(wid * b_per_w, b_per_w)], idx_v)

        def fire(g, buf, sem):
            for r in range(GB):
                for (off, n) in row_chunks:
                    pltpu.async_copy(
                        table_hbm.at[
                            idx_v.at[pl.ds(g * (GB * seq) + r * seq + off, n)]
                        ],
                        buf.at[r, pl.ds(off, n)],
                        sem,
                    )

        def drain(buf, sem):
            pltpu.make_async_copy(out_hbm.at[pl.ds(0, GB)], buf, sem).wait()

        def store(g, buf):
            pltpu.sync_copy(buf, out_hbm.at[pl.ds(wid * bw + g * GB, GB)])

        fire(0, buf_a, sem_a)

        def body(i, carry):
            g0 = 2 * i
            g1 = g0 + 1
            fire(g1, buf_b, sem_b)
            drain(buf_a, sem_a)
            store(g0, buf_a)
            fire(jnp.minimum(g0 + 2, ng - 1), buf_a, sem_a)
            drain(buf_b, sem_b)
            store(g1, buf_b)
            return carry

        lax.fori_loop(0, ng // 2, body, 0)
        drain(buf_a, sem_a)

    return emb_kernel


def kernel(glove_weight, word_indices):
    batch, seq = word_indices.shape
    info = plsc.get_sparse_core_info()
    nc, ns = info.num_cores, info.num_subcores
    return _make_kernel(batch, seq, nc, ns)(
        glove_weight, word_indices.reshape(batch * seq)
    )
